# baseline (speedup 1.0000x reference)
import functools

import jax
import jax.numpy as jnp
from jax import lax
from jax.experimental import pallas as pl
from jax.experimental.pallas import tpu as pltpu
from jax.experimental.pallas import tpu_sc as plsc

_F = 26
_V1 = 100001
_VS = 100008
_VP = 100352
_B = 16384
_NC, _NS, _L = 2, 16, 16
_CHUNK = 2048
_NBLK = _B // _CHUNK

_mesh = plsc.VectorSubcoreMesh(core_axis_name="c", subcore_axis_name="s")


@functools.partial(
    pl.kernel,
    out_type=jax.ShapeDtypeStruct((_F * _B,), jnp.float32),
    mesh=_mesh,
    scratch_types=[
        pltpu.VMEM((_VS,), jnp.float32),
        pltpu.VMEM((_B,), jnp.int32),
        pltpu.VMEM((2, _CHUNK), jnp.float32),
        pltpu.SemaphoreType.DMA,
        pltpu.SemaphoreType.DMA,
        pltpu.SemaphoreType.DMA,
    ],
    compiler_params=pltpu.CompilerParams(needs_layout_passes=False),
)
def _gather_fields(idx_hbm, tab_hbm, out_hbm, tab_v, idx_v, g_v, semt, semi, semo):
    fid = lax.axis_index("c") * _NS + lax.axis_index("s")

    @pl.when(fid < _F)
    def _work():
        row = pltpu.async_copy(tab_hbm.at[pl.ds(fid * _VP, _VS)], tab_v, semt)
        idx = pltpu.async_copy(idx_hbm.at[pl.ds(fid * _B, _B)], idx_v, semi)
        idx.wait()
        row.wait()

        outs = []
        for b in range(_NBLK):
            half = b % 2
            if len(outs) >= 2:
                outs[-2].wait()

            def _blk(c, carry, b=b, half=half):
                for u in range(8):
                    o = c * _L * 8 + u * _L
                    iv = idx_v[pl.ds(b * _CHUNK + o, _L)]
                    g_v[half, pl.ds(o, _L)] = plsc.load_gather(tab_v, [iv])
                return carry

            lax.fori_loop(0, _CHUNK // _L // 8, _blk, 0)
            outs.append(
                pltpu.async_copy(
                    g_v.at[half], out_hbm.at[pl.ds(fid * _B + b * _CHUNK, _CHUNK)], semo
                )
            )
        for cp in outs[-2:]:
            cp.wait()


def _combine_body(in_ref, out_ref):
    out_ref[...] = jnp.sum(in_ref[...], axis=0, keepdims=True)


_combine = pl.pallas_call(
    _combine_body,
    out_shape=jax.ShapeDtypeStruct((1, _B), jnp.float32),
)


def kernel(indices, tables):
    idx = indices.astype(jnp.int32)
    tab = jnp.pad(tables, ((0, 0), (0, _VP - _V1), (0, 0))).reshape(_F * _VP)
    gathered = _gather_fields(idx.reshape(_F * _B), tab)
    out = _combine(gathered.reshape(_F, _B))
    return out.reshape(_B, 1)


# speedup vs baseline: 1.0487x; 1.0487x over previous
import functools

import jax
import jax.numpy as jnp
from jax import lax
from jax.experimental import pallas as pl
from jax.experimental.pallas import tpu as pltpu
from jax.experimental.pallas import tpu_sc as plsc

_F = 26
_V1 = 100001
_VP = 100352
_B = 16384
_NC, _NS, _L = 2, 16, 16
_NW = _NC * _NS
_BPW = _B // _NW
_CH = _BPW // _L

_mesh = plsc.VectorSubcoreMesh(core_axis_name="c", subcore_axis_name="s")


@functools.partial(
    pl.kernel,
    out_type=jax.ShapeDtypeStruct((_B,), jnp.float32),
    mesh=_mesh,
    scratch_types=[
        pltpu.VMEM((_F * _BPW,), jnp.int32),
        pltpu.VMEM((_F * _BPW,), jnp.float32),
        pltpu.VMEM((_BPW,), jnp.float32),
        pltpu.SemaphoreType.DMA,
        pltpu.SemaphoreType.DMA,
    ],
    compiler_params=pltpu.CompilerParams(use_tc_tiling_on_sc=False),
)
def _lookup_sum(idx_hbm, tab_hbm, out_hbm, idx_v, gath_v, out_v, sem, sem2):
    wid = lax.axis_index("s") * _NC + lax.axis_index("c")
    base = wid * _BPW

    stage = [
        pltpu.async_copy(
            idx_hbm.at[pl.ds(f * _B + base, _BPW)],
            idx_v.at[pl.ds(f * _BPW, _BPW)],
            sem2,
        )
        for f in range(_F)
    ]
    copies = []
    for f in range(_F):
        stage[f].wait()
        sl = pl.ds(f * _BPW, _BPW)
        copies.append(pltpu.async_copy(tab_hbm.at[idx_v.at[sl]], gath_v.at[sl], sem))
    for cp in copies:
        cp.wait()

    def _acc(c, carry):
        s = gath_v[pl.ds(c * _L, _L)]
        for f in range(1, _F):
            s = s + gath_v[pl.ds(f * _BPW + c * _L, _L)]
        out_v[pl.ds(c * _L, _L)] = s
        return carry

    lax.fori_loop(0, _CH, _acc, 0)

    pltpu.sync_copy(out_v, out_hbm.at[pl.ds(base, _BPW)])


def kernel(indices, tables):
    idx = indices.astype(jnp.int32)
    offs = (jnp.arange(_F, dtype=jnp.int32) * _VP)[:, None]
    tab = jnp.pad(tables, ((0, 0), (0, _VP - _V1), (0, 0))).reshape(_F * _VP)
    out = _lookup_sum((idx + offs).reshape(_F * _B), tab)
    return out[:, None]
